# Initial kernel scaffold; baseline (speedup 1.0000x reference)
import jax
import jax.numpy as jnp
from jax.experimental import pallas as pl


def kernel(states, action_vec, W_edge, b_edge, W_node, b_node):
    raise NotImplementedError("write your pallas kernel here")


# speedup vs baseline: 24.4781x; 24.4781x over previous
import jax
import jax.numpy as jnp
from jax.experimental import pallas as pl

B = 2048
N = 5
D = 64
H = 64
A = 16
PAIRS = [(i, j) for i in range(N) for j in range(N) if i != j]
P = len(PAIRS)

BB = 256


def _gnn_kernel(states_ref, act_ref, We_ref, be_ref, Wn_ref, bn_ref, out_ref):
    s = states_ref[...]
    a = act_ref[...]

    agg = [None] * N
    for p, (i, j) in enumerate(PAIRS):
        si = s[:, i * D:(i + 1) * D]
        sj = s[:, j * D:(j + 1) * D]
        edge_in = jnp.concatenate([si, sj], axis=1)
        m = jnp.tanh(
            jnp.dot(edge_in, We_ref[p], preferred_element_type=jnp.float32)
            + be_ref[p]
        )
        agg[i] = m if agg[i] is None else agg[i] + m

    for n in range(N):
        node_in = jnp.concatenate(
            [s[:, n * D:(n + 1) * D], a[:, n * A:(n + 1) * A], agg[n]], axis=1
        )
        o = jnp.tanh(
            jnp.dot(node_in, Wn_ref[n], preferred_element_type=jnp.float32)
            + bn_ref[n]
        )
        out_ref[:, n * D:(n + 1) * D] = o


def kernel(states, action_vec, W_edge, b_edge, W_node, b_node):
    s2 = states.reshape(B, N * D)
    a2 = action_vec.reshape(B, N * A)
    grid = (B // BB,)
    out = pl.pallas_call(
        _gnn_kernel,
        grid=grid,
        in_specs=[
            pl.BlockSpec((BB, N * D), lambda g: (g, 0)),
            pl.BlockSpec((BB, N * A), lambda g: (g, 0)),
            pl.BlockSpec((P, 2 * D, H), lambda g: (0, 0, 0)),
            pl.BlockSpec((P, H), lambda g: (0, 0)),
            pl.BlockSpec((N, D + A + H, D), lambda g: (0, 0, 0)),
            pl.BlockSpec((N, D), lambda g: (0, 0)),
        ],
        out_specs=pl.BlockSpec((BB, N * D), lambda g: (g, 0)),
        out_shape=jax.ShapeDtypeStruct((B, N * D), jnp.float32),
    )(s2, a2, W_edge, b_edge, W_node, b_node)
    return out.reshape(B, N, D)


# speedup vs baseline: 26.4732x; 1.0815x over previous
import jax
import jax.numpy as jnp
from jax.experimental import pallas as pl

B = 2048
N = 5
D = 64
H = 64
A = 16
PAIRS = [(i, j) for i in range(N) for j in range(N) if i != j]
P = len(PAIRS)

BB = 512


def _gnn_kernel(states_ref, act_ref, We_ref, be_ref, Wn_ref, bn_ref, out_ref):
    s = states_ref[...]
    a = act_ref[...]

    agg = [None] * N
    for p, (i, j) in enumerate(PAIRS):
        si = s[:, i * D:(i + 1) * D]
        sj = s[:, j * D:(j + 1) * D]
        edge_in = jnp.concatenate([si, sj], axis=1)
        m = jnp.tanh(
            jnp.dot(edge_in, We_ref[p], preferred_element_type=jnp.float32)
            + be_ref[p]
        )
        agg[i] = m if agg[i] is None else agg[i] + m

    for n in range(N):
        node_in = jnp.concatenate(
            [s[:, n * D:(n + 1) * D], a[:, n * A:(n + 1) * A], agg[n]], axis=1
        )
        o = jnp.tanh(
            jnp.dot(node_in, Wn_ref[n], preferred_element_type=jnp.float32)
            + bn_ref[n]
        )
        out_ref[:, n * D:(n + 1) * D] = o


def kernel(states, action_vec, W_edge, b_edge, W_node, b_node):
    s2 = states.reshape(B, N * D)
    a2 = action_vec.reshape(B, N * A)
    grid = (B // BB,)
    out = pl.pallas_call(
        _gnn_kernel,
        grid=grid,
        in_specs=[
            pl.BlockSpec((BB, N * D), lambda g: (g, 0)),
            pl.BlockSpec((BB, N * A), lambda g: (g, 0)),
            pl.BlockSpec((P, 2 * D, H), lambda g: (0, 0, 0)),
            pl.BlockSpec((P, H), lambda g: (0, 0)),
            pl.BlockSpec((N, D + A + H, D), lambda g: (0, 0, 0)),
            pl.BlockSpec((N, D), lambda g: (0, 0)),
        ],
        out_specs=pl.BlockSpec((BB, N * D), lambda g: (g, 0)),
        out_shape=jax.ShapeDtypeStruct((B, N * D), jnp.float32),
    )(s2, a2, W_edge, b_edge, W_node, b_node)
    return out.reshape(B, N, D)
